# baseline (speedup 1.0000x reference)
import functools

import jax
import jax.numpy as jnp
from jax.experimental import pallas as pl

_SLOPE = 0.2
_F = 128


def _mm_attn_body(x_ref, w_ref, a_ref, h_ref, s_ref):
    h = jnp.dot(x_ref[...], w_ref[...], preferred_element_type=jnp.float32)
    h_ref[...] = h
    s_ref[...] = jnp.dot(h, a_ref[...], preferred_element_type=jnp.float32)


@functools.partial(jax.jit, static_argnames=("block",))
def _mm_attn(x, W, a_pair, block=2048):
    n = x.shape[0]
    grid = (n // block,) if n % block == 0 else (pl.cdiv(n, block),)
    k = a_pair.shape[1]
    h, s = pl.pallas_call(
        _mm_attn_body,
        grid=grid,
        in_specs=[
            pl.BlockSpec((block, _F), lambda i: (i, 0)),
            pl.BlockSpec((_F, _F), lambda i: (0, 0)),
            pl.BlockSpec((_F, k), lambda i: (0, 0)),
        ],
        out_specs=[
            pl.BlockSpec((block, _F), lambda i: (i, 0)),
            pl.BlockSpec((block, k), lambda i: (i, 0)),
        ],
        out_shape=[
            jax.ShapeDtypeStruct((n, _F), jnp.float32),
            jax.ShapeDtypeStruct((n, k), jnp.float32),
        ],
    )(x, W, a_pair)
    return h, s


def _seg_softmax_agg(e, seg, h_src_vals, n):
    m = jax.ops.segment_max(e, seg, num_segments=n)
    ex = jnp.exp(e - m[seg])
    d = jax.ops.segment_sum(ex, seg, num_segments=n)
    num = jax.ops.segment_sum(ex[:, None] * h_src_vals, seg, n)
    d = jnp.where(d == 0, 1.0, d)
    return num / d[:, None]


def _ccaba(x, idx, W, a_s, a_d, n):
    a2 = jnp.stack([a_s, a_d], axis=1)
    h, s = _mm_attn(x, W, a2)
    row, col = idx[0], idx[1]
    e = jax.nn.leaky_relu(s[row, 0] + s[col, 1], _SLOPE)
    return _seg_softmax_agg(e, row, h[col], n)


def _ccabi_st(x_s, inc, Wst, ast, n_t):
    h_st, s = _mm_attn(x_s, Wst, ast[:, None])
    row, col = inc[0], inc[1]
    e_st = jax.nn.leaky_relu(s[row, 0], _SLOPE)
    return _seg_softmax_agg(e_st, col, h_st[row], n_t)


def _ccabi_ts(x_t, inc, Wts, ats, n_s):
    h_ts, s = _mm_attn(x_t, Wts, ats[:, None])
    row, col = inc[0], inc[1]
    e_ts = jax.nn.leaky_relu(s[col, 0], _SLOPE)
    return _seg_softmax_agg(e_ts, row, h_ts[col], n_s)


def kernel(x_0, x_1, x_2, params, adjacency_0, adjacency_1, coadjacency_2,
           incidence_1, incidence_2):
    p = params
    n0, n1, n2 = x_0.shape[0], x_1.shape[0], x_2.shape[0]
    x_0_to_0 = _ccaba(x_0, adjacency_0, p['l1_00_W'], p['l1_00_as'], p['l1_00_ad'], n0)
    x_0_to_1 = _ccabi_st(x_0, incidence_1, p['l1_01_Wst'], p['l1_01_ast'], n1)
    x_1_to_0 = _ccabi_ts(x_1, incidence_1, p['l1_01_Wts'], p['l1_01_ats'], n0)
    x_1_to_2 = _ccabi_st(x_1, incidence_2, p['l1_12_Wst'], p['l1_12_ast'], n2)
    x_2_to_1 = _ccabi_ts(x_2, incidence_2, p['l1_12_Wts'], p['l1_12_ats'], n1)
    x_0_l1 = jax.nn.sigmoid(x_0_to_0 + x_1_to_0)
    x_1_l1 = jax.nn.sigmoid(x_0_to_1 + x_2_to_1)
    x_2_l1 = x_1_to_2
    x_0_to_0_b = _ccaba(x_0_l1, adjacency_0, p['l2_00_W'], p['l2_00_as'], p['l2_00_ad'], n0)
    x_0_to_1_b = _ccabi_st(x_0_l1, incidence_1, p['l2_01_Wst'], p['l2_01_ast'], n1)
    x_1_to_1_b = _ccaba(x_1_l1, adjacency_1, p['l2_11_W'], p['l2_11_as'], p['l2_11_ad'], n1)
    x_1_to_2_b = _ccabi_st(x_1_l1, incidence_2, p['l2_12_Wst'], p['l2_12_ast'], n2)
    x_2_to_2_b = _ccaba(x_2_l1, coadjacency_2, p['l2_22_W'], p['l2_22_as'], p['l2_22_ad'], n2)
    x_0_l2 = x_0_to_0_b
    x_1_l2 = jax.nn.sigmoid(x_0_to_1_b + x_1_to_1_b)
    x_2_l2 = jax.nn.sigmoid(x_1_to_2_b + x_2_to_2_b)
    return (x_0_l2, x_1_l2, x_2_l2)
